# Initial kernel scaffold; baseline (speedup 1.0000x reference)
import jax
import jax.numpy as jnp
from jax.experimental import pallas as pl


def kernel(adj, seq, W, bias, alpha):
    raise NotImplementedError("write your pallas kernel here")


# speedup vs baseline: 1.0422x; 1.0422x over previous
import jax
import jax.numpy as jnp
from jax.experimental import pallas as pl
from jax.experimental.pallas import tpu as pltpu


def _gcn_body(adj_ref, seq_ref, w_ref, b_ref, alpha_ref, out_ref):
    acc = jnp.dot(
        adj_ref[...], seq_ref[...], preferred_element_type=jnp.float32
    )
    o = jax.lax.dot_general(
        acc,
        w_ref[...],
        (((1,), (1,)), ((), ())),
        preferred_element_type=jnp.float32,
    )
    o = o + b_ref[...]
    out_ref[...] = jnp.where(o >= 0.0, o, alpha_ref[0, 0] * o)


def kernel(adj, seq, W, bias, alpha):
    B, M, K = adj.shape
    D = W.shape[0]
    adj2 = adj.reshape(M, K)
    seq2 = seq.reshape(K, seq.shape[2])
    bias2 = bias.reshape(1, D)
    alpha2 = jnp.asarray(alpha, jnp.float32).reshape(1, 1)

    TM = 400 if M % 400 == 0 else M
    grid = (M // TM,)

    out = pl.pallas_call(
        _gcn_body,
        grid=grid,
        in_specs=[
            pl.BlockSpec((TM, K), lambda m: (m, 0)),
            pl.BlockSpec((K, D), lambda m: (0, 0)),
            pl.BlockSpec((D, D), lambda m: (0, 0)),
            pl.BlockSpec((1, D), lambda m: (0, 0)),
            pl.BlockSpec((1, 1), lambda m: (0, 0)),
        ],
        out_specs=pl.BlockSpec((TM, D), lambda m: (m, 0)),
        out_shape=jax.ShapeDtypeStruct((M, D), jnp.float32),
        compiler_params=pltpu.CompilerParams(
            dimension_semantics=("parallel",),
        ),
    )(adj2, seq2, W, bias2, alpha2)
    return out.reshape(B, M, D)
